# baseline (speedup 1.0000x reference)
import jax
import jax.numpy as jnp
from jax import lax
from jax.experimental import pallas as pl
from jax.experimental.pallas import tpu as pltpu
from jax.experimental.pallas import tpu_sc as plsc

_B, _L, _D = 64, 256, 1024
_HALF = 128
_MARGINS = (0.2, 0.4)
_NSUB = 16
_LANES = 16
_ROWS_PER_TILE = _B // _NSUB


def _sim_body(t_ref, l_ref, sim_ref):
    t = t_ref[...]
    lbl = l_ref[...]
    tn = t / jnp.maximum(jnp.sqrt(jnp.sum(t * t, axis=-1, keepdims=True)), 1e-12)
    ln = lbl / jnp.maximum(jnp.sqrt(jnp.sum(lbl * lbl, axis=-1, keepdims=True)), 1e-12)
    sim_ref[...] = jax.lax.dot_general(
        tn, ln, (((1,), (1,)), ((), ())),
        preferred_element_type=jnp.float32,
        precision=jax.lax.Precision.HIGHEST,
    )


def _sc_mine_body(sim_hbm, tgt_hbm, out_hbm, part_hbm, sim_v, tgt_v, vec_v, red_v):
    sid = lax.axis_index("s")
    cid = lax.axis_index("c")
    iota = lax.iota(jnp.int32, _LANES)

    pltpu.sync_copy(sim_hbm.at[pl.ds(sid * _ROWS_PER_TILE, _ROWS_PER_TILE)], sim_v)
    pltpu.sync_copy(tgt_hbm.at[pl.ds(sid * _ROWS_PER_TILE, _ROWS_PER_TILE)], tgt_v)

    zeros = jnp.zeros((_LANES,), jnp.float32)
    acc_s = [zeros, zeros]
    acc_c = [zeros, zeros]
    for r in range(_ROWS_PER_TILE):
        for lvl in range(2):
            base = lvl * _HALF
            chunks = [sim_v[r, pl.ds(base + _LANES * k, _LANES)]
                      for k in range(_HALF // _LANES)]
            mv = chunks[0]
            for k in range(1, len(chunks)):
                mv = jnp.maximum(mv, chunks[k])
            m = jnp.max(mv)
            s0 = jnp.sum(jnp.where(iota == 0, chunks[0], 0.0))
            s1 = jnp.sum(jnp.where(iota == 1, chunks[0], 0.0))
            margin = _MARGINS[lvl]
            for k in range(len(chunks)):
                ck = chunks[k]
                tk = tgt_v[r, pl.ds(base + _LANES * k, _LANES)]
                if k == 0:
                    fb = jnp.where(iota == 0, s1, s0)
                else:
                    fb = s0
                negv = jnp.where(ck < m, m, fb)
                per = jnp.maximum(negv - ck + margin, 0.0)
                vm = jnp.where(tk != 0, 1.0, 0.0)
                acc_s[lvl] = acc_s[lvl] + per * vm
                acc_c[lvl] = acc_c[lvl] + vm

    part = (jnp.where(iota == 0, jnp.sum(acc_s[0]), 0.0)
            + jnp.where(iota == 1, jnp.sum(acc_c[0]), 0.0)
            + jnp.where(iota == 2, jnp.sum(acc_s[1]), 0.0)
            + jnp.where(iota == 3, jnp.sum(acc_c[1]), 0.0))
    vec_v[...] = part
    pltpu.sync_copy(vec_v, part_hbm.at[cid * _NSUB + sid])
    plsc.subcore_barrier()

    @pl.when(sid == 0)
    def _():
        pltpu.sync_copy(part_hbm.at[pl.ds(cid * _NSUB, _NSUB)], red_v)
        acc = red_v[0, :]
        for i in range(1, _NSUB):
            acc = acc + red_v[i, :]
        big_s1 = jnp.sum(jnp.where(iota == 0, acc, 0.0))
        big_c1 = jnp.sum(jnp.where(iota == 1, acc, 0.0))
        big_s2 = jnp.sum(jnp.where(iota == 2, acc, 0.0))
        big_c2 = jnp.sum(jnp.where(iota == 3, acc, 0.0))
        nv = (jnp.where(iota == 0, big_s1, 0.0)
              + jnp.where(iota == 1, big_s2, 0.0))
        c2v = jnp.where(iota >= 0, big_c2, 0.0)
        dv = jnp.where(iota == 0, big_c1, jnp.maximum(c2v, 1.0))
        q = nv / dv
        gate = jnp.where(
            (iota == 0) | ((iota == 1) & (c2v >= 3.0)), 1.0, 0.0)
        loss = jnp.sum(q * gate)
        vec_v[...] = jnp.where(iota == 0, loss, 0.0)
        pltpu.sync_copy(vec_v, out_hbm.at[cid])


def kernel(text_embed, label_embed, target):
    tgt = target.astype(jnp.int32)
    sim = pl.pallas_call(
        _sim_body,
        out_shape=jax.ShapeDtypeStruct((_B, _L), jnp.float32),
    )(text_embed, label_embed)

    mine = pl.kernel(
        _sc_mine_body,
        out_type=(jax.ShapeDtypeStruct((2, _LANES), jnp.float32),
                  jax.ShapeDtypeStruct((2 * _NSUB, _LANES), jnp.float32)),
        mesh=plsc.VectorSubcoreMesh(
            core_axis_name="c", subcore_axis_name="s",
            num_cores=2, num_subcores=_NSUB),
        compiler_params=pltpu.CompilerParams(needs_layout_passes=False),
        scratch_types=[
            pltpu.VMEM((_ROWS_PER_TILE, _L), jnp.float32),
            pltpu.VMEM((_ROWS_PER_TILE, _L), jnp.int32),
            pltpu.VMEM((_LANES,), jnp.float32),
            pltpu.VMEM((_NSUB, _LANES), jnp.float32),
        ],
    )
    out, _ = mine(sim, tgt)
    return out[0, 0]
